# Initial kernel scaffold; baseline (speedup 1.0000x reference)
import jax
import jax.numpy as jnp
from jax.experimental import pallas as pl


def kernel(pred, target):
    raise NotImplementedError("write your pallas kernel here")


# speedup vs baseline: 6.4643x; 6.4643x over previous
import jax
import jax.numpy as jnp
from jax.experimental import pallas as pl
from jax.experimental.pallas import tpu as pltpu


def _dwt_l1_body(p_ref, t_ref, out_ref):
    w2 = p_ref.shape[-1]
    w = w2 // 2
    p = p_ref[...]
    t = t_ref[...]
    et = p[:, :w] - t[:, :w]
    eb = p[:, w:] - t[:, w:]
    av = jnp.abs(et + eb)
    aw = jnp.abs(et - eb)
    avr = jnp.concatenate([av[:, 1:], av[:, :1]], axis=1)
    awr = jnp.concatenate([aw[:, 1:], aw[:, :1]], axis=1)
    f = jnp.maximum(av, avr) + jnp.maximum(aw, awr)
    colsum = jnp.sum(f, axis=0, keepdims=True)
    lane = jax.lax.broadcasted_iota(jnp.int32, colsum.shape, 1)
    masked = jnp.where((lane & 1) == 0, colsum, 0.0)
    out_ref[...] = jnp.sum(masked, axis=1, keepdims=True)[None]


def kernel(pred, target):
    B, C, H, W = pred.shape
    S = B * C
    x = pred.reshape(S, H // 2, 2 * W)
    y = target.reshape(S, H // 2, 2 * W)

    spec = pl.BlockSpec((None, H // 2, 2 * W), lambda i: (i, 0, 0))

    partials = pl.pallas_call(
        _dwt_l1_body,
        grid=(S,),
        in_specs=[spec, spec],
        out_specs=pl.BlockSpec((1, 1, 1), lambda i: (i, 0, 0)),
        out_shape=jax.ShapeDtypeStruct((S, 1, 1), jnp.float32),
        compiler_params=pltpu.CompilerParams(
            dimension_semantics=("parallel",),
        ),
    )(x, y)

    n = S * (H // 2) * (W // 2)
    return jnp.sum(partials) * (1.0 / n)


# speedup vs baseline: 16.9745x; 2.6259x over previous
import jax
import jax.numpy as jnp
from jax.experimental import pallas as pl
from jax.experimental.pallas import tpu as pltpu


def _dwt_l1_body(p_ref, t_ref, out_ref):
    h, w = p_ref.shape
    e = p_ref[...] - t_ref[...]
    e_dn = jnp.concatenate([e[1:], e[:1]], axis=0)
    av = jnp.abs(e + e_dn)
    aw = jnp.abs(e - e_dn)
    avr = jnp.concatenate([av[:, 1:], av[:, :1]], axis=1)
    awr = jnp.concatenate([aw[:, 1:], aw[:, :1]], axis=1)
    f = jnp.maximum(av, avr) + jnp.maximum(aw, awr)
    row = jax.lax.broadcasted_iota(jnp.int32, (h, w), 0)
    f = jnp.where((row & 1) == 0, f, 0.0)
    colsum = jnp.sum(f, axis=0, keepdims=True)
    lane = jax.lax.broadcasted_iota(jnp.int32, colsum.shape, 1)
    masked = jnp.where((lane & 1) == 0, colsum, 0.0)
    out_ref[...] = jnp.sum(masked, axis=1, keepdims=True)[None, None]


def kernel(pred, target):
    B, C, H, W = pred.shape

    spec = pl.BlockSpec((None, None, H, W), lambda b, c: (b, c, 0, 0))

    partials = pl.pallas_call(
        _dwt_l1_body,
        grid=(B, C),
        in_specs=[spec, spec],
        out_specs=pl.BlockSpec((1, 1, 1, 1), lambda b, c: (b, c, 0, 0)),
        out_shape=jax.ShapeDtypeStruct((B, C, 1, 1), jnp.float32),
        compiler_params=pltpu.CompilerParams(
            dimension_semantics=("parallel", "parallel"),
        ),
    )(pred, target)

    n = B * C * (H // 2) * (W // 2)
    return jnp.sum(partials) * (1.0 / n)


# speedup vs baseline: 23.9358x; 1.4101x over previous
import jax
import jax.numpy as jnp
from jax.experimental import pallas as pl
from jax.experimental.pallas import tpu as pltpu


def _dwt_l1_body(p_ref, t_ref, out_ref):
    w = p_ref.shape[-1]
    e = (p_ref[...] - t_ref[...]).reshape(-1, w)
    h = e.shape[0]
    e_dn = jnp.concatenate([e[1:], e[:1]], axis=0)
    av = jnp.abs(e + e_dn)
    aw = jnp.abs(e - e_dn)
    avr = jnp.concatenate([av[:, 1:], av[:, :1]], axis=1)
    awr = jnp.concatenate([aw[:, 1:], aw[:, :1]], axis=1)
    f = jnp.maximum(av, avr) + jnp.maximum(aw, awr)
    row = jax.lax.broadcasted_iota(jnp.int32, (h, w), 0)
    f = jnp.where((row & 1) == 0, f, 0.0)
    colsum = jnp.sum(f, axis=0, keepdims=True)
    lane = jax.lax.broadcasted_iota(jnp.int32, colsum.shape, 1)
    masked = jnp.where((lane & 1) == 0, colsum, 0.0)
    out_ref[...] = jnp.sum(masked, axis=1, keepdims=True)[None, None]


def kernel(pred, target):
    B, C, H, W = pred.shape

    BB = 1
    spec = pl.BlockSpec((BB, C, H, W), lambda i: (i, 0, 0, 0))

    partials = pl.pallas_call(
        _dwt_l1_body,
        grid=(B // BB,),
        in_specs=[spec, spec],
        out_specs=pl.BlockSpec((1, 1, 1, 1), lambda i: (i, 0, 0, 0)),
        out_shape=jax.ShapeDtypeStruct((B // BB, 1, 1, 1), jnp.float32),
        compiler_params=pltpu.CompilerParams(
            dimension_semantics=("parallel",),
        ),
    )(pred, target)

    n = B * C * (H // 2) * (W // 2)
    return jnp.sum(partials) * (1.0 / n)


# speedup vs baseline: 25.9150x; 1.0827x over previous
import jax
import jax.numpy as jnp
from jax.experimental import pallas as pl
from jax.experimental.pallas import tpu as pltpu


def _dwt_l1_body(p_ref, t_ref, out_ref):
    w = p_ref.shape[-1]
    e = (p_ref[...] - t_ref[...]).reshape(-1, w)
    h = e.shape[0]
    e_dn = jnp.concatenate([e[1:], e[:1]], axis=0)
    av = jnp.abs(e + e_dn)
    aw = jnp.abs(e - e_dn)
    avr = jnp.concatenate([av[:, 1:], av[:, :1]], axis=1)
    awr = jnp.concatenate([aw[:, 1:], aw[:, :1]], axis=1)
    f = jnp.maximum(av, avr) + jnp.maximum(aw, awr)
    row = jax.lax.broadcasted_iota(jnp.int32, (h, w), 0)
    f = jnp.where((row & 1) == 0, f, 0.0)
    colsum = jnp.sum(f, axis=0, keepdims=True)
    lane = jax.lax.broadcasted_iota(jnp.int32, colsum.shape, 1)
    masked = jnp.where((lane & 1) == 0, colsum, 0.0)
    out_ref[...] = jnp.sum(masked, axis=1, keepdims=True)[None, None]


def kernel(pred, target):
    B, C, H, W = pred.shape

    BB = 2
    spec = pl.BlockSpec((BB, C, H, W), lambda i: (i, 0, 0, 0))

    partials = pl.pallas_call(
        _dwt_l1_body,
        grid=(B // BB,),
        in_specs=[spec, spec],
        out_specs=pl.BlockSpec((1, 1, 1, 1), lambda i: (i, 0, 0, 0)),
        out_shape=jax.ShapeDtypeStruct((B // BB, 1, 1, 1), jnp.float32),
        compiler_params=pltpu.CompilerParams(
            dimension_semantics=("parallel",),
        ),
    )(pred, target)

    n = B * C * (H // 2) * (W // 2)
    return jnp.sum(partials) * (1.0 / n)
